# baseline (speedup 1.0000x reference)
import functools

import jax
import jax.numpy as jnp
from jax import lax
from jax.experimental import pallas as pl
from jax.experimental.pallas import tpu as pltpu
from jax.experimental.pallas import tpu_sc as plsc

_NUM_CORES = 2
_NUM_SUBCORES = 16
_NW = _NUM_CORES * _NUM_SUBCORES
_CHUNK = 128


def kernel(input, table):
    B, S = input.shape
    V, D = table.shape
    total = B * S
    per_w = total // _NW
    n_chunks = per_w // _CHUNK
    nbuf = 4
    n_groups = n_chunks // nbuf

    idx = input.T.reshape(_NW, n_chunks, _CHUNK)
    tablewide = jnp.concatenate([table, table], axis=1)

    mesh = plsc.VectorSubcoreMesh(core_axis_name="c", subcore_axis_name="s")

    @functools.partial(
        pl.kernel,
        mesh=mesh,
        out_type=jax.ShapeDtypeStruct(
            (_NW, n_chunks, _CHUNK, 2 * D), jnp.float32
        ),
        scratch_types=(
            [pltpu.VMEM((n_chunks, _CHUNK), jnp.int32)]
            + [pltpu.VMEM((_CHUNK, 2 * D), jnp.float32) for _ in range(nbuf)]
            + [pltpu.SemaphoreType.DMA for _ in range(2 * nbuf)]
        ),
        compiler_params=pltpu.CompilerParams(use_tc_tiling_on_sc=True),
    )
    def gather_kernel(idx_hbm, table_hbm, out_hbm, idx_v, *bufs_and_sems):
        rows = bufs_and_sems[:nbuf]
        gsem = bufs_and_sems[nbuf : 2 * nbuf]
        osem = bufs_and_sems[2 * nbuf :]
        wid = lax.axis_index("s") * _NUM_CORES + lax.axis_index("c")
        pltpu.sync_copy(idx_hbm.at[wid], idx_v)

        def body(j, carry):
            pltpu.async_copy(
                table_hbm.at[idx_v.at[j]], rows[0], gsem[0]
            ).wait()
            pltpu.async_copy(rows[0], out_hbm.at[wid, j], osem[0]).wait()
            return carry

        lax.fori_loop(0, n_chunks, body, 0)

    out = gather_kernel(idx, tablewide)
    return out[..., :D].reshape(S, B, D).transpose(1, 0, 2)


# speedup vs baseline: 1.0626x; 1.0626x over previous
import functools

import jax
import jax.numpy as jnp
from jax import lax
from jax.experimental import pallas as pl
from jax.experimental.pallas import tpu as pltpu
from jax.experimental.pallas import tpu_sc as plsc

_NUM_CORES = 2
_NUM_SUBCORES = 16
_NW = _NUM_CORES * _NUM_SUBCORES
_CHUNK = 128


def kernel(input, table):
    B, S = input.shape
    V, D = table.shape
    total = B * S
    per_w = total // _NW
    n_chunks = per_w // _CHUNK
    nbuf = 2
    n_groups = n_chunks // nbuf

    idx = input.T.reshape(_NW, n_chunks, _CHUNK)
    tablewide = jnp.concatenate([table, table], axis=1)

    mesh = plsc.VectorSubcoreMesh(core_axis_name="c", subcore_axis_name="s")

    @functools.partial(
        pl.kernel,
        mesh=mesh,
        out_type=jax.ShapeDtypeStruct(
            (_NW, n_chunks, _CHUNK, 2 * D), jnp.float32
        ),
        scratch_types=(
            [pltpu.VMEM((n_chunks, _CHUNK), jnp.int32)]
            + [pltpu.VMEM((_CHUNK, 2 * D), jnp.float32) for _ in range(nbuf)]
            + [pltpu.SemaphoreType.DMA for _ in range(2 * nbuf)]
        ),
        compiler_params=pltpu.CompilerParams(use_tc_tiling_on_sc=True),
    )
    def gather_kernel(idx_hbm, table_hbm, out_hbm, idx_v, *bufs_and_sems):
        rows = bufs_and_sems[:nbuf]
        gsem = bufs_and_sems[nbuf : 2 * nbuf]
        osem = bufs_and_sems[2 * nbuf :]
        wid = lax.axis_index("s") * _NUM_CORES + lax.axis_index("c")
        pltpu.sync_copy(idx_hbm.at[wid], idx_v)

        pltpu.async_copy(table_hbm.at[idx_v.at[0]], rows[0], gsem[0])

        def group(g, carry):
            for b in range(nbuf):
                j = g * nbuf + b
                pltpu.make_async_copy(
                    table_hbm.at[idx_v.at[j]], rows[b], gsem[b]
                ).wait()

                @pl.when(j + 1 < n_chunks)
                def _():
                    nb = (b + 1) % nbuf
                    pltpu.async_copy(
                        table_hbm.at[idx_v.at[j + 1]], rows[nb], gsem[nb]
                    )

                pltpu.async_copy(
                    rows[b], out_hbm.at[wid, j], osem[b]
                ).wait()
            return carry

        lax.fori_loop(0, n_chunks // nbuf, group, 0)

    out = gather_kernel(idx, tablewide)
    return out[..., :D].reshape(S, B, D).transpose(1, 0, 2)


# speedup vs baseline: 1.2987x; 1.2222x over previous
import functools

import jax
import jax.numpy as jnp
from jax import lax
from jax.experimental import pallas as pl
from jax.experimental.pallas import tpu as pltpu
from jax.experimental.pallas import tpu_sc as plsc

_NUM_CORES = 2
_NUM_SUBCORES = 16
_NW = _NUM_CORES * _NUM_SUBCORES
_CHUNK = 128


def kernel(input, table):
    B, S = input.shape
    V, D = table.shape
    total = B * S
    per_w = total // _NW
    n_chunks = per_w // _CHUNK
    nbuf = 4
    n_groups = n_chunks // nbuf

    idx = input.T.reshape(_NW, n_chunks, _CHUNK)
    tablewide = jnp.pad(table, ((0, 0), (0, D)))

    mesh = plsc.VectorSubcoreMesh(core_axis_name="c", subcore_axis_name="s")

    @functools.partial(
        pl.kernel,
        mesh=mesh,
        out_type=jax.ShapeDtypeStruct(
            (_NW, n_chunks, _CHUNK, 2 * D), jnp.float32
        ),
        scratch_types=(
            [pltpu.VMEM((n_chunks, _CHUNK), jnp.int32)]
            + [pltpu.VMEM((_CHUNK, 2 * D), jnp.float32) for _ in range(nbuf)]
            + [pltpu.SemaphoreType.DMA for _ in range(2 * nbuf)]
        ),
        compiler_params=pltpu.CompilerParams(use_tc_tiling_on_sc=True),
    )
    def gather_kernel(idx_hbm, table_hbm, out_hbm, idx_v, *bufs_and_sems):
        rows = bufs_and_sems[:nbuf]
        gsem = bufs_and_sems[nbuf : 2 * nbuf]
        osem = bufs_and_sems[2 * nbuf :]
        wid = lax.axis_index("s") * _NUM_CORES + lax.axis_index("c")
        pltpu.sync_copy(idx_hbm.at[wid], idx_v)

        ahead = nbuf - 1
        for b in range(ahead):
            pltpu.async_copy(table_hbm.at[idx_v.at[b]], rows[b], gsem[b])

        def group(g, carry):
            for b in range(nbuf):
                j = g * nbuf + b
                pltpu.make_async_copy(
                    table_hbm.at[idx_v.at[j]], rows[b], gsem[b]
                ).wait()

                @pl.when(j + ahead < n_chunks)
                def _():
                    nb = (b + ahead) % nbuf
                    pltpu.async_copy(
                        table_hbm.at[idx_v.at[j + ahead]], rows[nb], gsem[nb]
                    )

                pltpu.async_copy(
                    rows[b], out_hbm.at[wid, j], osem[b]
                ).wait()
            return carry

        lax.fori_loop(0, n_chunks // nbuf, group, 0)

    out = gather_kernel(idx, tablewide)
    return out[..., :D].reshape(S, B, D).transpose(1, 0, 2)


# speedup vs baseline: 1.3041x; 1.0042x over previous
import functools

import jax
import jax.numpy as jnp
from jax import lax
from jax.experimental import pallas as pl
from jax.experimental.pallas import tpu as pltpu
from jax.experimental.pallas import tpu_sc as plsc

_NUM_CORES = 2
_NUM_SUBCORES = 16
_NW = _NUM_CORES * _NUM_SUBCORES
_CHUNK = 128


def kernel(input, table):
    B, S = input.shape
    V, D = table.shape
    total = B * S
    per_w = total // _NW
    n_chunks = per_w // _CHUNK
    nbuf = 5
    n_groups = n_chunks // nbuf

    idx = input.T.reshape(_NW, n_chunks, _CHUNK)
    tablewide = jnp.pad(table, ((0, 0), (0, D)))

    mesh = plsc.VectorSubcoreMesh(core_axis_name="c", subcore_axis_name="s")

    @functools.partial(
        pl.kernel,
        mesh=mesh,
        out_type=jax.ShapeDtypeStruct(
            (_NW, n_chunks, _CHUNK, 2 * D), jnp.float32
        ),
        scratch_types=(
            [pltpu.VMEM((n_chunks, _CHUNK), jnp.int32)]
            + [pltpu.VMEM((_CHUNK, 2 * D), jnp.float32) for _ in range(nbuf)]
            + [pltpu.SemaphoreType.DMA for _ in range(2 * nbuf)]
        ),
        compiler_params=pltpu.CompilerParams(use_tc_tiling_on_sc=True),
    )
    def gather_kernel(idx_hbm, table_hbm, out_hbm, idx_v, *bufs_and_sems):
        rows = bufs_and_sems[:nbuf]
        gsem = bufs_and_sems[nbuf : 2 * nbuf]
        osem = bufs_and_sems[2 * nbuf :]
        wid = lax.axis_index("s") * _NUM_CORES + lax.axis_index("c")
        pltpu.sync_copy(idx_hbm.at[wid], idx_v)

        ahead = nbuf - 1
        for b in range(ahead):
            pltpu.async_copy(table_hbm.at[idx_v.at[b]], rows[b], gsem[b])

        def group(g, carry):
            for b in range(nbuf):
                j = g * nbuf + b
                pltpu.make_async_copy(
                    table_hbm.at[idx_v.at[j]], rows[b], gsem[b]
                ).wait()

                @pl.when(j + ahead < n_chunks)
                def _():
                    nb = (b + ahead) % nbuf
                    pltpu.async_copy(
                        table_hbm.at[idx_v.at[j + ahead]], rows[nb], gsem[nb]
                    )

                pltpu.async_copy(
                    rows[b], out_hbm.at[wid, j], osem[b]
                ).wait()
            return carry

        lax.fori_loop(0, n_chunks // nbuf, group, 0)

    out = gather_kernel(idx, tablewide)
    return out[..., :D].reshape(S, B, D).transpose(1, 0, 2)
